# baseline (speedup 1.0000x reference)
import jax
import jax.numpy as jnp
from jax.experimental import pallas as pl
from jax.experimental.pallas import tpu as pltpu

_ROWS = 8


def _mask_body(x_ref, m_ref, o_ref):
    o_ref[...] = jnp.where(m_ref[...] != 0, x_ref[...], jnp.float32(0))


def kernel(inputs, mask):
    b, u = inputs.shape
    m2 = mask.reshape(1, u).astype(jnp.int8)
    grid = (b // _ROWS,)
    return pl.pallas_call(
        _mask_body,
        grid=grid,
        in_specs=[
            pl.BlockSpec((_ROWS, u), lambda i: (i, 0)),
            pl.BlockSpec((1, u), lambda i: (0, 0)),
        ],
        out_specs=pl.BlockSpec((_ROWS, u), lambda i: (i, 0)),
        out_shape=jax.ShapeDtypeStruct((b, u), inputs.dtype),
        compiler_params=pltpu.CompilerParams(
            dimension_semantics=("parallel",),
        ),
    )(inputs, m2)


# speedup vs baseline: 1.4278x; 1.4278x over previous
import jax
import jax.numpy as jnp
from jax.experimental import pallas as pl
from jax.experimental.pallas import tpu as pltpu

_ROWS = 32


def _mask_body(x_ref, m_ref, o_ref):
    o_ref[...] = jnp.where(m_ref[...] != 0, x_ref[...], jnp.float32(0))


def kernel(inputs, mask):
    b, u = inputs.shape
    m2 = mask.reshape(1, u).astype(jnp.int8)
    grid = (b // _ROWS,)
    return pl.pallas_call(
        _mask_body,
        grid=grid,
        in_specs=[
            pl.BlockSpec((_ROWS, u), lambda i: (i, 0)),
            pl.BlockSpec((1, u), lambda i: (0, 0)),
        ],
        out_specs=pl.BlockSpec((_ROWS, u), lambda i: (i, 0)),
        out_shape=jax.ShapeDtypeStruct((b, u), inputs.dtype),
        compiler_params=pltpu.CompilerParams(
            dimension_semantics=("parallel",),
        ),
    )(inputs, m2)


# speedup vs baseline: 1.6430x; 1.1507x over previous
import jax
import jax.numpy as jnp
from jax.experimental import pallas as pl
from jax.experimental.pallas import tpu as pltpu

_ROWS = 64


def _mask_body(x_ref, m_ref, o_ref):
    o_ref[...] = jnp.where(m_ref[...] != 0, x_ref[...], jnp.float32(0))


def kernel(inputs, mask):
    b, u = inputs.shape
    m2 = mask.reshape(1, u).astype(jnp.int8)
    grid = (b // _ROWS,)
    return pl.pallas_call(
        _mask_body,
        grid=grid,
        in_specs=[
            pl.BlockSpec((_ROWS, u), lambda i: (i, 0)),
            pl.BlockSpec((1, u), lambda i: (0, 0)),
        ],
        out_specs=pl.BlockSpec((_ROWS, u), lambda i: (i, 0)),
        out_shape=jax.ShapeDtypeStruct((b, u), inputs.dtype),
        compiler_params=pltpu.CompilerParams(
            dimension_semantics=("parallel",),
        ),
    )(inputs, m2)
